# baseline (speedup 1.0000x reference)
import jax
import jax.numpy as jnp
from jax import lax
from jax.experimental import pallas as pl
from jax.experimental.pallas import tpu as pltpu
from jax.experimental.pallas import tpu_sc as plsc

B = 16384
DIM = 16
PACK = 8
NC = 2
NS = 16
NW = NC * NS
BPW = B // NW
CHUNK = 128
NCHUNK = BPW // CHUNK
NGRP = CHUNK // DIM


def _body(u_hbm, i_hbm, n_hbm, user_hbm, item_hbm, pos_hbm, neg_hbm,
          idx_u, idx_i, idx_n, idx8_u, idx8_i, idx8_n,
          bufs_u, bufs_i, bufs_n, pos_v, neg_v, sem0, sem1):
    wid = lax.axis_index("s") * NC + lax.axis_index("c")

    pltpu.sync_copy(u_hbm.at[pl.ds(wid * NCHUNK, NCHUNK)], idx_u)
    pltpu.sync_copy(i_hbm.at[pl.ds(wid * NCHUNK, NCHUNK)], idx_i)
    pltpu.sync_copy(n_hbm.at[pl.ds(wid * NCHUNK, NCHUNK)], idx_n)

    for raw, shifted in ((idx_u, idx8_u), (idx_i, idx8_i), (idx_n, idx8_n)):
        for j in range(NCHUNK):
            for k in range(CHUNK // 16):
                sl = pl.ds(k * 16, 16)
                shifted[j, sl] = lax.shift_right_logical(
                    raw[j, sl], jnp.full((16,), 3, jnp.int32))

    sems = (sem0, sem1)

    def fire(j):
        buf = j % 2
        s = sems[buf]
        return (
            pltpu.async_copy(user_hbm.at[idx8_u.at[j]], bufs_u.at[buf], s),
            pltpu.async_copy(item_hbm.at[idx8_i.at[j]], bufs_i.at[buf], s),
            pltpu.async_copy(item_hbm.at[idx8_n.at[j]], bufs_n.at[buf], s),
        )

    inflight = [fire(0), fire(1)]

    for j in range(NCHUNK):
        buf = j % 2
        for c in inflight[j]:
            c.wait()
        bu, bi, bn = bufs_u.at[buf], bufs_i.at[buf], bufs_n.at[buf]
        for g in range(NGRP):
            rows = g * DIM + lax.iota(jnp.int32, 16)
            gsl = pl.ds(g * DIM, 16)
            seven = jnp.full((16,), 7, jnp.int32)
            off_u = jnp.multiply(jnp.bitwise_and(idx_u[j, gsl], seven),
                            jnp.full((16,), DIM, jnp.int32))
            off_i = jnp.multiply(jnp.bitwise_and(idx_i[j, gsl], seven),
                            jnp.full((16,), DIM, jnp.int32))
            off_n = jnp.multiply(jnp.bitwise_and(idx_n[j, gsl], seven),
                            jnp.full((16,), DIM, jnp.int32))
            accp = jnp.zeros((16,), jnp.float32)
            accn = jnp.zeros((16,), jnp.float32)
            for d in range(DIM):
                dd = jnp.full((16,), d, jnp.int32)
                uc = plsc.load_gather(bu, [rows, off_u + dd])
                ic = plsc.load_gather(bi, [rows, off_i + dd])
                nc = plsc.load_gather(bn, [rows, off_n + dd])
                accp = accp + uc * ic
                accn = accn + uc * nc
            pos_v[pl.ds(j * CHUNK + g * DIM, 16)] = accp
            neg_v[pl.ds(j * CHUNK + g * DIM, 16)] = accn
        if j + 2 < NCHUNK:
            inflight.append(fire(j + 2))

    pltpu.sync_copy(pos_v, pos_hbm.at[pl.ds(wid * BPW, BPW)])
    pltpu.sync_copy(neg_v, neg_hbm.at[pl.ds(wid * BPW, BPW)])


@jax.jit
def kernel(u, i, neg_i, user_emb, item_emb):
    u2 = u.astype(jnp.int32).reshape(NW * NCHUNK, CHUNK)
    i2 = i.astype(jnp.int32).reshape(NW * NCHUNK, CHUNK)
    n2 = neg_i.astype(jnp.int32).reshape(NW * NCHUNK, CHUNK)
    user_p = user_emb.reshape(user_emb.shape[0] // PACK, PACK * DIM)
    item_p = item_emb.reshape(item_emb.shape[0] // PACK, PACK * DIM)

    mesh = plsc.VectorSubcoreMesh(core_axis_name="c", subcore_axis_name="s",
                                  num_cores=NC, num_subcores=NS)
    run = pl.kernel(
        _body,
        out_type=(jax.ShapeDtypeStruct((B,), jnp.float32),
                  jax.ShapeDtypeStruct((B,), jnp.float32)),
        mesh=mesh,
        scratch_types=[
            pltpu.VMEM((NCHUNK, CHUNK), jnp.int32),
            pltpu.VMEM((NCHUNK, CHUNK), jnp.int32),
            pltpu.VMEM((NCHUNK, CHUNK), jnp.int32),
            pltpu.VMEM((NCHUNK, CHUNK), jnp.int32),
            pltpu.VMEM((NCHUNK, CHUNK), jnp.int32),
            pltpu.VMEM((NCHUNK, CHUNK), jnp.int32),
            pltpu.VMEM((2, CHUNK, PACK * DIM), jnp.float32),
            pltpu.VMEM((2, CHUNK, PACK * DIM), jnp.float32),
            pltpu.VMEM((2, CHUNK, PACK * DIM), jnp.float32),
            pltpu.VMEM((BPW,), jnp.float32),
            pltpu.VMEM((BPW,), jnp.float32),
            pltpu.SemaphoreType.DMA,
            pltpu.SemaphoreType.DMA,
        ],
        compiler_params=pltpu.CompilerParams(needs_layout_passes=False,
                                             use_tc_tiling_on_sc=True),
    )
    return run(u2, i2, n2, user_p, item_p)
